# baseline (speedup 1.0000x reference)
import functools

import jax
import jax.numpy as jnp
from jax.experimental import pallas as pl
from jax.experimental.pallas import tpu as pltpu


def _matmul_kernel(x_ref, w_ref, b_ref, o_ref):
    xt = x_ref[...].astype(w_ref.dtype)
    acc = jnp.dot(xt, w_ref[...], preferred_element_type=jnp.float32)
    o_ref[...] = (acc + b_ref[...]).astype(o_ref.dtype)


@functools.partial(jax.jit, static_argnames=("tt",))
def _embed(x, w_t, b_row, tt):
    B, T, C = x.shape
    _, D = w_t.shape
    w_bf = w_t.astype(jnp.bfloat16)

    cost = pl.CostEstimate(
        flops=2 * B * T * C * D,
        transcendentals=0,
        bytes_accessed=(B * T * C + B * T * D) * 4 + C * D * 2 + 4 * D,
    )
    y_flat = pl.pallas_call(
        _matmul_kernel,
        out_shape=jax.ShapeDtypeStruct((T, B * D), x.dtype),
        grid=(B, pl.cdiv(T, tt)),
        in_specs=[
            pl.BlockSpec((None, tt, C), lambda b, ti: (b, ti, 0)),
            pl.BlockSpec((C, D), lambda b, ti: (0, 0)),
            pl.BlockSpec((1, D), lambda b, ti: (0, 0)),
        ],
        out_specs=pl.BlockSpec((tt, D), lambda b, ti: (ti, b)),
        compiler_params=pltpu.CompilerParams(
            dimension_semantics=("parallel", "parallel"),
            vmem_limit_bytes=96 * 1024 * 1024,
        ),
        cost_estimate=cost,
    )(x, w_bf, b_row)
    return y_flat.reshape(T, B, D)


def kernel(x, w_t, b_row):
    return _embed(x, w_t, b_row, tt=256)


# speedup vs baseline: 3.1618x; 3.1618x over previous
import functools

import jax
import jax.numpy as jnp
from jax.experimental import pallas as pl
from jax.experimental.pallas import tpu as pltpu


def _embed_kernel(x_ref, w_ref, b_ref, o_ref, *, bb, tt):
    C = x_ref.shape[2]
    D = w_ref.shape[1]
    xt = x_ref[...].reshape(bb * tt, C).astype(jnp.bfloat16)
    w = w_ref[...].astype(jnp.bfloat16)
    acc = jnp.dot(xt, w, preferred_element_type=jnp.float32)
    acc = acc + b_ref[...]
    o_ref[...] = jnp.swapaxes(acc.reshape(bb, tt, D), 0, 1).astype(o_ref.dtype)


@functools.partial(jax.jit, static_argnames=("bb", "tt"))
def _embed(x, w_t, b_row, bb, tt):
    B, T, C = x.shape
    _, D = w_t.shape

    cost = pl.CostEstimate(
        flops=2 * B * T * C * D,
        transcendentals=0,
        bytes_accessed=(B * T * C + B * T * D) * 4 + C * D * 4 + 4 * D,
    )
    y = pl.pallas_call(
        functools.partial(_embed_kernel, bb=bb, tt=tt),
        out_shape=jax.ShapeDtypeStruct((T, B, D), x.dtype),
        grid=(B // bb, pl.cdiv(T, tt)),
        in_specs=[
            pl.BlockSpec((bb, tt, C), lambda bi, ti: (bi, ti, 0)),
            pl.BlockSpec((C, D), lambda bi, ti: (0, 0)),
            pl.BlockSpec((1, D), lambda bi, ti: (0, 0)),
        ],
        out_specs=pl.BlockSpec((tt, bb, D), lambda bi, ti: (ti, bi, 0)),
        compiler_params=pltpu.CompilerParams(
            dimension_semantics=("parallel", "parallel"),
            vmem_limit_bytes=100 * 1024 * 1024,
        ),
        cost_estimate=cost,
    )(x, w_t, b_row)
    return y


def kernel(x, w_t, b_row):
    return _embed(x, w_t, b_row, bb=8, tt=256)


# speedup vs baseline: 3.5462x; 1.1216x over previous
import functools

import jax
import jax.numpy as jnp
from jax.experimental import pallas as pl
from jax.experimental.pallas import tpu as pltpu


def _embed_kernel(x_ref, w_ref, b_ref, o_ref, *, bb, tt):
    C = x_ref.shape[2]
    D = w_ref.shape[1]
    xt = jnp.swapaxes(x_ref[...].astype(jnp.bfloat16), 0, 1)
    xt = xt.reshape(tt * bb, C)
    w = w_ref[...].astype(jnp.bfloat16)
    acc = jnp.dot(xt, w, preferred_element_type=jnp.float32)
    acc = acc + b_ref[...]
    o_ref[...] = acc.reshape(tt, bb, D).astype(o_ref.dtype)


@functools.partial(jax.jit, static_argnames=("bb", "tt"))
def _embed(x, w_t, b_row, bb, tt):
    B, T, C = x.shape
    _, D = w_t.shape

    cost = pl.CostEstimate(
        flops=2 * B * T * C * D,
        transcendentals=0,
        bytes_accessed=(B * T * C + B * T * D) * 4 + C * D * 4 + 4 * D,
    )
    y = pl.pallas_call(
        functools.partial(_embed_kernel, bb=bb, tt=tt),
        out_shape=jax.ShapeDtypeStruct((T, B, D), x.dtype),
        grid=(B // bb, pl.cdiv(T, tt)),
        in_specs=[
            pl.BlockSpec((bb, tt, C), lambda bi, ti: (bi, ti, 0)),
            pl.BlockSpec((C, D), lambda bi, ti: (0, 0)),
            pl.BlockSpec((1, D), lambda bi, ti: (0, 0)),
        ],
        out_specs=pl.BlockSpec((tt, bb, D), lambda bi, ti: (ti, bi, 0)),
        compiler_params=pltpu.CompilerParams(
            dimension_semantics=("parallel", "parallel"),
            vmem_limit_bytes=100 * 1024 * 1024,
        ),
        cost_estimate=cost,
    )(x, w_t, b_row)
    return y


def kernel(x, w_t, b_row):
    return _embed(x, w_t, b_row, bb=16, tt=128)
